# Initial kernel scaffold; baseline (speedup 1.0000x reference)
import jax
import jax.numpy as jnp
from jax.experimental import pallas as pl


def kernel(x_locs, x_clusters, edge_ll, edge_lc_src, edge_lc_dst, edge_cc, W_l, W_r, b_l, W2_l, W2_r, b2, W3, b3):
    raise NotImplementedError("write your pallas kernel here")


# speedup vs baseline: 12.8923x; 12.8923x over previous
import functools

import jax
import jax.numpy as jnp
from jax import lax
from jax.experimental import pallas as pl
from jax.experimental.pallas import tpu as pltpu
from jax.experimental.pallas import tpu_sc as plsc

F32 = jnp.float32
I32 = jnp.int32

N_LOC = 10000
N_CLU = 1000
D = 128
W_ROW = 16

NC = 2
NS = 16
NW = NC * NS
CHUNK = 128

LL_CHUNKS = 80
LL_EDGES_PAD = NW * LL_CHUNKS * CHUNK
LL_ROWS = 10240
LC_CHUNKS = 3
LC_EDGES_PAD = NW * LC_CHUNKS * CHUNK
CLU_ROWS = 1024
CC_CHUNKS = 4
CC_EDGES_PAD = NW * CC_CHUNKS * CHUNK


def _tc_gemm_body(x_ref, w_ref, b_ref, p_ref, r_ref):
    res = jnp.dot(x_ref[...], w_ref[...], preferred_element_type=F32)
    res = res + b_ref[...]
    p_ref[...] = res[:, :W_ROW]
    r_ref[...] = res[:, W_ROW:]


def _tc_gemm(x, w, b, n_rows):
    return pl.pallas_call(
        _tc_gemm_body,
        out_shape=[jax.ShapeDtypeStruct((n_rows, W_ROW), F32)] * 2,
    )(x, w, b)


def _tc_cluster_body(xc_ref, cmt_ref, wx_ref, wc_ref, b_ref, z_ref, rc_ref):
    cm = jnp.max(cmt_ref[...], axis=0)
    res = jnp.dot(xc_ref[...], wx_ref[...], preferred_element_type=F32)
    res = res + jnp.dot(cm, wc_ref[...], preferred_element_type=F32)
    res = res + b_ref[...]
    z_ref[...] = res[:, :W_ROW]
    rc_ref[...] = res[:, W_ROW:]


def _tc_cluster(xc, cmt, wx, wc, b):
    return pl.pallas_call(
        _tc_cluster_body,
        out_shape=[jax.ShapeDtypeStruct((CLU_ROWS, W_ROW), F32)] * 2,
    )(xc, cmt, wx, wc, b)


def _tc_final_body(a_ref, b_ref, rc_ref, w3_ref, b3_ref, o_ref):
    s = a_ref[...] + b_ref[...]
    cnt = s[:, 4:5]
    mean = s[:, :4] / jnp.maximum(cnt, 1.0)
    o = mean + rc_ref[...][:, :4]
    rows = lax.broadcasted_iota(I32, (CLU_ROWS, 4), 0)
    o = jnp.where(rows < N_CLU, o, -3.0e38)
    m = jnp.max(o, axis=0, keepdims=True)
    prod = m * w3_ref[...]
    o_ref[...] = jnp.sum(prod, axis=1, keepdims=True) + b3_ref[...]


def _tc_final(a2a, a2b, rc, w3t, b3m):
    return pl.pallas_call(
        _tc_final_body,
        out_shape=jax.ShapeDtypeStruct((1, 1), F32),
    )(a2a, a2b, rc, w3t, b3m)


_MESH = plsc.VectorSubcoreMesh(core_axis_name="c", subcore_axis_name="s")


def _make_scatter_add(n_chunks, tab_rows, table_rows):
    nbuf = min(4, n_chunks)
    n_groups = n_chunks // nbuf
    assert n_groups * nbuf == n_chunks
    zr = tab_rows // NS

    @functools.partial(
        pl.kernel,
        out_type=[jax.ShapeDtypeStruct((tab_rows, W_ROW), F32)] * 2,
        mesh=_MESH,
        compiler_params=pltpu.CompilerParams(use_tc_tiling_on_sc=False),
        scratch_types=(
            [pltpu.VMEM((n_chunks, CHUNK), I32)] * 2
            + [pltpu.VMEM((CHUNK, W_ROW), F32)] * nbuf
            + [pltpu.VMEM((zr, W_ROW), F32)]
            + [pltpu.VMEM_SHARED((tab_rows, W_ROW), F32)]
            + [pltpu.SemaphoreType.DMA] * (2 * nbuf)
        ),
    )
    def sck(table_hbm, zeros_hbm, src_hbm, dst_hbm, acc_a, acc_b, *rest):
        src_v, dst_v = rest[0], rest[1]
        bufs = rest[2:2 + nbuf]
        bb = rest[2 + nbuf]
        acc_sh = rest[3 + nbuf]
        gsem = rest[4 + nbuf:4 + 2 * nbuf]
        ssem = rest[4 + 2 * nbuf:4 + 3 * nbuf]

        cid = lax.axis_index("c")
        sid = lax.axis_index("s")
        wid = cid * NS + sid

        pltpu.sync_copy(src_hbm.at[pl.ds(wid * n_chunks, n_chunks)], src_v)
        pltpu.sync_copy(dst_hbm.at[pl.ds(wid * n_chunks, n_chunks)], dst_v)
        pltpu.sync_copy(zeros_hbm.at[pl.ds(sid * zr, zr)], bb)
        pltpu.sync_copy(bb, acc_sh.at[pl.ds(sid * zr, zr)])
        plsc.subcore_barrier()

        for b in range(nbuf):
            pltpu.async_copy(table_hbm.at[src_v.at[b]], bufs[b], gsem[b])

        def group(g, carry):
            for b in range(nbuf):
                j = g * nbuf + b
                pltpu.make_async_copy(
                    table_hbm.at[src_v.at[j]], bufs[b], gsem[b]).wait()
                pltpu.async_copy(
                    bufs[b], acc_sh.at[dst_v.at[j]], ssem[b], add=True)
            for b in range(nbuf):
                j = g * nbuf + b
                pltpu.make_async_copy(
                    bufs[b], acc_sh.at[dst_v.at[j]], ssem[b]).wait()
                jn = (g + 1) * nbuf + b

                @pl.when(jn < n_chunks)
                def _():
                    pltpu.async_copy(
                        table_hbm.at[src_v.at[jn]], bufs[b], gsem[b])
            return carry

        lax.fori_loop(0, n_groups, group, 0)

        plsc.subcore_barrier()
        pltpu.sync_copy(acc_sh.at[pl.ds(sid * zr, zr)], bb)

        @pl.when(cid == 0)
        def _():
            pltpu.sync_copy(bb, acc_a.at[pl.ds(sid * zr, zr)])

        @pl.when(cid == 1)
        def _():
            pltpu.sync_copy(bb, acc_b.at[pl.ds(sid * zr, zr)])

    return sck


_scatter_add_ll = _make_scatter_add(LL_CHUNKS, LL_ROWS, N_LOC)
_scatter_add_cc = _make_scatter_add(CC_CHUNKS, CLU_ROWS, CLU_ROWS)

_TAB_W = CLU_ROWS * W_ROW


@functools.partial(
    pl.kernel,
    out_type=jax.ShapeDtypeStruct((NW, _TAB_W), F32),
    mesh=_MESH,
    compiler_params=pltpu.CompilerParams(use_tc_tiling_on_sc=False),
    scratch_types=(
        [pltpu.VMEM((LC_CHUNKS, CHUNK), I32)]
        + [pltpu.VMEM((LC_CHUNKS * CHUNK,), I32)]
        + [pltpu.VMEM((CHUNK, W_ROW), F32)] * 3
        + [pltpu.VMEM((_TAB_W,), F32)]
        + [pltpu.SemaphoreType.DMA] * 3
    ),
)
def _segmax_lc(acc_a, acc_b, r_hbm, zeros_hbm, src_hbm, dst_hbm, out_hbm,
               src_v, dst_v, g0, g1, g2, tab_v, s0, s1, s2):
    cid = lax.axis_index("c")
    sid = lax.axis_index("s")
    wid = cid * NS + sid

    pltpu.sync_copy(src_hbm.at[pl.ds(wid * LC_CHUNKS, LC_CHUNKS)], src_v)
    pltpu.sync_copy(
        dst_hbm.at[pl.ds(wid * LC_CHUNKS * CHUNK, LC_CHUNKS * CHUNK)], dst_v)
    pltpu.sync_copy(zeros_hbm, tab_v)

    for j in range(LC_CHUNKS):
        c0 = pltpu.async_copy(acc_a.at[src_v.at[j]], g0, s0)
        c1 = pltpu.async_copy(acc_b.at[src_v.at[j]], g1, s1)
        c2 = pltpu.async_copy(r_hbm.at[src_v.at[j]], g2, s2)
        c0.wait()
        c1.wait()
        c2.wait()

        def body(q, carry):
            dvec = dst_v[pl.ds(j * CHUNK + q * 16, 16)]
            for l in range(16):
                r = q * 16 + l
                s = g0[r, :] + g1[r, :]
                cnt = s[7]
                el = jnp.maximum(s / jnp.maximum(cnt, 1.0) + g2[r, :], 0.0)
                off = dvec[l] * W_ROW
                tab_v[pl.ds(off, 16)] = jnp.maximum(
                    tab_v[pl.ds(off, 16)], el)
            return carry

        lax.fori_loop(0, CHUNK // 16, body, 0)

    pltpu.sync_copy(tab_v, out_hbm.at[wid])


def _pad_edges(src, dst, total, pad_dst):
    n = src.shape[0]
    src = jnp.pad(src.astype(I32), (0, total - n))
    dst = jnp.pad(dst.astype(I32), (0, total - n), constant_values=pad_dst)
    return src.reshape(-1, CHUNK), dst

def kernel(x_locs, x_clusters, edge_ll, edge_lc_src, edge_lc_dst, edge_cc,
           W_l, W_r, b_l, W2_l, W2_r, b2, W3, b3):
    w1 = jnp.zeros((D, 2 * W_ROW), F32)
    w1 = w1.at[:, 0:7].set(W_l).at[:, W_ROW:W_ROW + 7].set(W_r)
    b1 = jnp.zeros((1, 2 * W_ROW), F32)
    b1 = b1.at[0, 7].set(1.0).at[0, W_ROW:W_ROW + 7].set(b_l)

    w2x = jnp.zeros((D, 2 * W_ROW), F32)
    w2x = w2x.at[:, 0:4].set(W2_l[:D]).at[:, W_ROW:W_ROW + 4].set(W2_r[:D])
    w2c = jnp.zeros((W_ROW, 2 * W_ROW), F32)
    w2c = w2c.at[0:7, 0:4].set(W2_l[D:]).at[0:7, W_ROW:W_ROW + 4].set(W2_r[D:])
    b2v = jnp.zeros((1, 2 * W_ROW), F32)
    b2v = b2v.at[0, 4].set(1.0).at[0, W_ROW:W_ROW + 4].set(b2)

    P, R = _tc_gemm(x_locs, w1, b1, N_LOC)

    src_ll, dst_ll = _pad_edges(edge_ll[0], edge_ll[1], LL_EDGES_PAD, N_LOC)
    acc_a, acc_b = _scatter_add_ll(
        P, jnp.zeros((LL_ROWS, W_ROW), F32), src_ll,
        dst_ll.reshape(-1, CHUNK))

    src_lc, dst_lc = _pad_edges(edge_lc_src, edge_lc_dst, LC_EDGES_PAD, N_CLU)
    tabs = _segmax_lc(acc_a, acc_b, R, jnp.zeros((_TAB_W,), F32),
                      src_lc, dst_lc)
    cmt = tabs.reshape(NW, CLU_ROWS, W_ROW)

    xc = jnp.zeros((CLU_ROWS, D), F32).at[:N_CLU].set(x_clusters)
    Z, Rc = _tc_cluster(xc, cmt, w2x, w2c, b2v)

    src_cc, dst_cc = _pad_edges(edge_cc[0], edge_cc[1], CC_EDGES_PAD, N_CLU)
    a2a, a2b = _scatter_add_cc(
        Z, jnp.zeros((CLU_ROWS, W_ROW), F32), src_cc,
        dst_cc.reshape(-1, CHUNK))

    out = _tc_final(a2a, a2b, Rc, W3.reshape(1, 4), b3.reshape(1, 1))
    return out.reshape(1)


# speedup vs baseline: 13.4483x; 1.0431x over previous
import functools

import jax
import jax.numpy as jnp
from jax import lax
from jax.experimental import pallas as pl
from jax.experimental.pallas import tpu as pltpu
from jax.experimental.pallas import tpu_sc as plsc

F32 = jnp.float32
I32 = jnp.int32

N_LOC = 10000
N_CLU = 1000
D = 128
W_ROW = 16

NC = 2
NS = 16
NW = NC * NS
CHUNK = 128

LL_CHUNKS = 80
LL_EDGES_PAD = NW * LL_CHUNKS * CHUNK
LL_ROWS = 10240
LC_CHUNKS = 3
LC_EDGES_PAD = NW * LC_CHUNKS * CHUNK
CLU_ROWS = 1024
CC_CHUNKS = 4
CC_EDGES_PAD = NW * CC_CHUNKS * CHUNK


def _tc_gemm_body(x_ref, w_ref, b_ref, p_ref, r_ref):
    res = jnp.dot(x_ref[...], w_ref[...], preferred_element_type=F32)
    res = res + b_ref[...]
    p_ref[...] = res[:, :W_ROW]
    r_ref[...] = res[:, W_ROW:]


def _tc_gemm(x, w, b, n_rows):
    return pl.pallas_call(
        _tc_gemm_body,
        out_shape=[jax.ShapeDtypeStruct((n_rows, W_ROW), F32)] * 2,
    )(x, w, b)


def _tc_cluster_body(xc_ref, cmt_ref, wx_ref, wc_ref, b_ref, z_ref, rc_ref):
    cm = jnp.max(cmt_ref[...], axis=0)
    res = jnp.dot(xc_ref[...], wx_ref[...], preferred_element_type=F32)
    res = res + jnp.dot(cm, wc_ref[...], preferred_element_type=F32)
    res = res + b_ref[...]
    z_ref[...] = res[:, :W_ROW]
    rc_ref[...] = res[:, W_ROW:]


def _tc_cluster(xc, cmt, wx, wc, b):
    return pl.pallas_call(
        _tc_cluster_body,
        out_shape=[jax.ShapeDtypeStruct((CLU_ROWS, W_ROW), F32)] * 2,
    )(xc, cmt, wx, wc, b)


def _tc_final_body(a_ref, b_ref, rc_ref, w3_ref, b3_ref, o_ref):
    s = a_ref[...] + b_ref[...]
    cnt = s[:, 4:5]
    mean = s[:, :4] / jnp.maximum(cnt, 1.0)
    o = mean + rc_ref[...][:, :4]
    rows = lax.broadcasted_iota(I32, (CLU_ROWS, 4), 0)
    o = jnp.where(rows < N_CLU, o, -3.0e38)
    m = jnp.max(o, axis=0, keepdims=True)
    prod = m * w3_ref[...]
    o_ref[...] = jnp.sum(prod, axis=1, keepdims=True) + b3_ref[...]


def _tc_final(a2a, a2b, rc, w3t, b3m):
    return pl.pallas_call(
        _tc_final_body,
        out_shape=jax.ShapeDtypeStruct((1, 1), F32),
    )(a2a, a2b, rc, w3t, b3m)


_MESH = plsc.VectorSubcoreMesh(core_axis_name="c", subcore_axis_name="s")


def _make_scatter_add(n_chunks, tab_rows, table_rows):
    nslot = min(8, n_chunks)
    nair = nslot // 2
    n_groups = n_chunks // nslot
    assert n_groups * nslot == n_chunks
    zr = tab_rows // NS

    @functools.partial(
        pl.kernel,
        out_type=[jax.ShapeDtypeStruct((tab_rows, W_ROW), F32)] * 2,
        mesh=_MESH,
        compiler_params=pltpu.CompilerParams(use_tc_tiling_on_sc=False),
        scratch_types=(
            [pltpu.VMEM((n_chunks, CHUNK), I32)] * 2
            + [pltpu.VMEM((CHUNK, W_ROW), F32)] * nslot
            + [pltpu.VMEM_SHARED((tab_rows, W_ROW), F32)]
            + [pltpu.SemaphoreType.DMA] * (2 * nslot + 2)
        ),
    )
    def sck(table_hbm, zeros_hbm, src_hbm, dst_hbm, acc_a, acc_b, *rest):
        src_v, dst_v = rest[0], rest[1]
        bufs = rest[2:2 + nslot]
        acc_sh = rest[2 + nslot]
        gsem = rest[3 + nslot:3 + 2 * nslot]
        ssem = rest[3 + 2 * nslot:3 + 3 * nslot]
        isem = rest[3 + 3 * nslot]
        zsem = rest[4 + 3 * nslot]

        cid = lax.axis_index("c")
        sid = lax.axis_index("s")
        wid = cid * NS + sid

        ic0 = pltpu.async_copy(
            src_hbm.at[pl.ds(wid * n_chunks, n_chunks)], src_v, isem)
        ic1 = pltpu.async_copy(
            dst_hbm.at[pl.ds(wid * n_chunks, n_chunks)], dst_v, isem)
        zc = pltpu.async_copy(
            zeros_hbm.at[pl.ds(sid * zr, zr)],
            acc_sh.at[pl.ds(sid * zr, zr)], zsem)
        ic0.wait()
        ic1.wait()
        zc.wait()
        plsc.subcore_barrier()

        def gather(j, b):
            pltpu.async_copy(table_hbm.at[src_v.at[j]], bufs[b], gsem[b])

        def scatter(j, b):
            pltpu.async_copy(
                bufs[b], acc_sh.at[dst_v.at[j]], ssem[b], add=True)

        def wait_gather(b):
            pltpu.make_async_copy(
                table_hbm.at[src_v.at[0]], bufs[b], gsem[b]).wait()

        def wait_scatter(b):
            pltpu.make_async_copy(
                bufs[b], acc_sh.at[dst_v.at[0]], ssem[b]).wait()

        for b in range(nair):
            gather(b, b)

        def group(g, carry):
            for b in range(nslot):
                j = g * nslot + b
                wait_gather(b)
                scatter(j, b)
                bn = (b + nair) % nslot
                jn = j + nair

                @pl.when(jn >= nslot)
                def _():
                    wait_scatter(bn)

                @pl.when(jn < n_chunks)
                def _():
                    gather(jn, bn)
            return carry

        lax.fori_loop(0, n_groups, group, 0)

        for j in range(n_chunks - nair, n_chunks):
            wait_scatter(j % nslot)

        plsc.subcore_barrier()

        @pl.when(cid == 0)
        def _():
            pltpu.sync_copy(
                acc_sh.at[pl.ds(sid * zr, zr)], acc_a.at[pl.ds(sid * zr, zr)])

        @pl.when(cid == 1)
        def _():
            pltpu.sync_copy(
                acc_sh.at[pl.ds(sid * zr, zr)], acc_b.at[pl.ds(sid * zr, zr)])

    return sck


_scatter_add_ll = _make_scatter_add(LL_CHUNKS, LL_ROWS, N_LOC)
_scatter_add_cc = _make_scatter_add(CC_CHUNKS, CLU_ROWS, CLU_ROWS)

_TAB_W = CLU_ROWS * W_ROW


@functools.partial(
    pl.kernel,
    out_type=jax.ShapeDtypeStruct((NW, _TAB_W), F32),
    mesh=_MESH,
    compiler_params=pltpu.CompilerParams(use_tc_tiling_on_sc=False),
    scratch_types=(
        [pltpu.VMEM((LC_CHUNKS, CHUNK), I32)]
        + [pltpu.VMEM((LC_CHUNKS * CHUNK,), I32)]
        + [pltpu.VMEM((CHUNK, W_ROW), F32)] * 3
        + [pltpu.VMEM((_TAB_W,), F32)]
        + [pltpu.SemaphoreType.DMA] * 3
    ),
)
def _segmax_lc(acc_a, acc_b, r_hbm, zeros_hbm, src_hbm, dst_hbm, out_hbm,
               src_v, dst_v, g0, g1, g2, tab_v, s0, s1, s2):
    cid = lax.axis_index("c")
    sid = lax.axis_index("s")
    wid = cid * NS + sid

    pltpu.sync_copy(src_hbm.at[pl.ds(wid * LC_CHUNKS, LC_CHUNKS)], src_v)
    pltpu.sync_copy(
        dst_hbm.at[pl.ds(wid * LC_CHUNKS * CHUNK, LC_CHUNKS * CHUNK)], dst_v)
    pltpu.sync_copy(zeros_hbm, tab_v)

    for j in range(LC_CHUNKS):
        c0 = pltpu.async_copy(acc_a.at[src_v.at[j]], g0, s0)
        c1 = pltpu.async_copy(acc_b.at[src_v.at[j]], g1, s1)
        c2 = pltpu.async_copy(r_hbm.at[src_v.at[j]], g2, s2)
        c0.wait()
        c1.wait()
        c2.wait()

        def body(q, carry):
            dvec = dst_v[pl.ds(j * CHUNK + q * 16, 16)]
            for l in range(16):
                r = q * 16 + l
                s = g0[r, :] + g1[r, :]
                cnt = s[7]
                el = jnp.maximum(s / jnp.maximum(cnt, 1.0) + g2[r, :], 0.0)
                off = dvec[l] * W_ROW
                tab_v[pl.ds(off, 16)] = jnp.maximum(
                    tab_v[pl.ds(off, 16)], el)
            return carry

        lax.fori_loop(0, CHUNK // 16, body, 0)

    pltpu.sync_copy(tab_v, out_hbm.at[wid])


def _pad_edges(src, dst, total, pad_dst):
    n = src.shape[0]
    src = jnp.pad(src.astype(I32), (0, total - n))
    dst = jnp.pad(dst.astype(I32), (0, total - n), constant_values=pad_dst)
    return src.reshape(-1, CHUNK), dst

def kernel(x_locs, x_clusters, edge_ll, edge_lc_src, edge_lc_dst, edge_cc,
           W_l, W_r, b_l, W2_l, W2_r, b2, W3, b3):
    w1 = jnp.zeros((D, 2 * W_ROW), F32)
    w1 = w1.at[:, 0:7].set(W_l).at[:, W_ROW:W_ROW + 7].set(W_r)
    b1 = jnp.zeros((1, 2 * W_ROW), F32)
    b1 = b1.at[0, 7].set(1.0).at[0, W_ROW:W_ROW + 7].set(b_l)

    w2x = jnp.zeros((D, 2 * W_ROW), F32)
    w2x = w2x.at[:, 0:4].set(W2_l[:D]).at[:, W_ROW:W_ROW + 4].set(W2_r[:D])
    w2c = jnp.zeros((W_ROW, 2 * W_ROW), F32)
    w2c = w2c.at[0:7, 0:4].set(W2_l[D:]).at[0:7, W_ROW:W_ROW + 4].set(W2_r[D:])
    b2v = jnp.zeros((1, 2 * W_ROW), F32)
    b2v = b2v.at[0, 4].set(1.0).at[0, W_ROW:W_ROW + 4].set(b2)

    P, R = _tc_gemm(x_locs, w1, b1, N_LOC)

    src_ll, dst_ll = _pad_edges(edge_ll[0], edge_ll[1], LL_EDGES_PAD, N_LOC)
    acc_a, acc_b = _scatter_add_ll(
        P, jnp.zeros((LL_ROWS, W_ROW), F32), src_ll,
        dst_ll.reshape(-1, CHUNK))

    src_lc, dst_lc = _pad_edges(edge_lc_src, edge_lc_dst, LC_EDGES_PAD, N_CLU)
    tabs = _segmax_lc(acc_a, acc_b, R, jnp.zeros((_TAB_W,), F32),
                      src_lc, dst_lc)
    cmt = tabs.reshape(NW, CLU_ROWS, W_ROW)

    xc = jnp.zeros((CLU_ROWS, D), F32).at[:N_CLU].set(x_clusters)
    Z, Rc = _tc_cluster(xc, cmt, w2x, w2c, b2v)

    src_cc, dst_cc = _pad_edges(edge_cc[0], edge_cc[1], CC_EDGES_PAD, N_CLU)
    a2a, a2b = _scatter_add_cc(
        Z, jnp.zeros((CLU_ROWS, W_ROW), F32), src_cc,
        dst_cc.reshape(-1, CHUNK))

    out = _tc_final(a2a, a2b, Rc, W3.reshape(1, 4), b3.reshape(1, 1))
    return out.reshape(1)
